# baseline (speedup 1.0000x reference)
import functools

import jax
import jax.numpy as jnp
from jax import lax
from jax.experimental import pallas as pl
from jax.experimental.pallas import tpu as pltpu

K = 1024
D = 64
JB = 1024


def _vq_block(x_ref, w_ref, q_ref, loss_ref, cnt_ref):
    xc = x_ref[0]
    w = w_ref[...]
    mm = lax.dot_general(xc, w, (((0,), (1,)), ((), ())),
                         preferred_element_type=jnp.float32)
    xsq = jnp.sum(xc * xc, axis=0)[:, None]
    wsq = jnp.sum(w * w, axis=1)
    d = (xsq + wsq[None, :]) - 2.0 * mm
    m = jnp.min(d, axis=1, keepdims=True)
    ks = lax.broadcasted_iota(jnp.int32, d.shape, 1)
    nearest = jnp.min(jnp.where(d == m, ks, K), axis=1)
    oh = (ks == nearest[:, None]).astype(jnp.float32)
    qc = lax.dot_general(w, oh, (((0,), (1,)), ((), ())),
                         preferred_element_type=jnp.float32)
    q_ref[0] = qc
    loss_ref[...] = jnp.broadcast_to(jnp.sum(m), (1, 1, 128))
    cnt_ref[...] = jnp.sum(oh, axis=0)[None, None, :]


def kernel(inputs, W, beta):
    B, C, H, Wd = inputs.shape
    N = B * H * Wd
    nb = N // JB
    x = inputs.reshape(B, C, H * Wd)
    q, lsum, cnt = pl.pallas_call(
        _vq_block,
        grid=(nb,),
        in_specs=[
            pl.BlockSpec((1, D, JB), lambda j: (j, 0, 0)),
            pl.BlockSpec((K, D), lambda j: (0, 0)),
        ],
        out_specs=[
            pl.BlockSpec((1, D, JB), lambda j: (j, 0, 0)),
            pl.BlockSpec((1, 1, 128), lambda j: (j, 0, 0)),
            pl.BlockSpec((1, 1, K), lambda j: (j, 0, 0)),
        ],
        out_shape=[
            jax.ShapeDtypeStruct((B, D, H * Wd), jnp.float32),
            jax.ShapeDtypeStruct((nb, 1, 128), jnp.float32),
            jax.ShapeDtypeStruct((nb, 1, K), jnp.float32),
        ],
    )(x, W)
    loss_mean = jnp.sum(lsum[:, 0, 0]) / (N * D)
    loss = loss_mean + beta * loss_mean
    e_mean = jnp.sum(cnt[:, 0, :], axis=0) / N
    perplexity = jnp.exp(-jnp.sum(e_mean * jnp.log(e_mean + 1e-10)))
    quantized_out = q.reshape(B, C, H, Wd)
    return (loss, quantized_out, perplexity)
